# baseline (speedup 1.0000x reference)
import functools

import jax
import jax.numpy as jnp
from jax import lax
from jax.experimental import pallas as pl
from jax.experimental.pallas import tpu as pltpu
from jax.experimental.pallas import tpu_sc as plsc

VOCAB = 50257
HIDDEN = 768
MAX_LEN = 2048
BATCH = 4
SEQ = 2048

_INFO = plsc.get_sparse_core_info()
_NC = _INFO.num_cores
_NS = _INFO.num_subcores
_NW = _NC * _NS
_SPW = SEQ // _NW
_VECS = HIDDEN // 16


def _emb_body(idx_hbm, word_hbm, pos_hbm, out_hbm, idx_v, rows_v, pos_v, sem):
    wid = lax.axis_index("s") * _NC + lax.axis_index("c")
    s0 = wid * _SPW

    pltpu.sync_copy(pos_hbm.at[pl.ds(s0, _SPW)], pos_v)

    for b in range(BATCH):
        row0 = b * SEQ + s0
        pltpu.sync_copy(idx_hbm.at[pl.ds(row0, _SPW)], idx_v)
        pltpu.async_copy(word_hbm.at[idx_v], rows_v, sem).wait()

        def add_body(i, _, rows_v=rows_v, pos_v=pos_v):
            r = i // _VECS
            c = (i % _VECS) * 16
            rows_v[r, pl.ds(c, 16)] = (
                rows_v[r, pl.ds(c, 16)] + pos_v[r, pl.ds(c, 16)]
            )
            return _

        lax.fori_loop(0, _SPW * _VECS, add_body, 0)
        pltpu.sync_copy(rows_v, out_hbm.at[pl.ds(row0, _SPW)])


@functools.partial(jax.jit, static_argnames=())
def _embed(idx_flat, word_table, pos_table):
    mesh = plsc.VectorSubcoreMesh(core_axis_name="c", subcore_axis_name="s")
    k = pl.kernel(
        _emb_body,
        out_type=jax.ShapeDtypeStruct((BATCH * SEQ, HIDDEN), jnp.float32),
        mesh=mesh,
        scratch_types=[
            pltpu.VMEM((_SPW,), jnp.int32),
            pltpu.VMEM((_SPW, HIDDEN), jnp.float32),
            pltpu.VMEM((_SPW, HIDDEN), jnp.float32),
            pltpu.SemaphoreType.DMA,
        ],
    )
    return k(idx_flat, word_table, pos_table)


def kernel(indices, word_table, pos_table):
    idx_flat = indices.reshape(-1)
    out = _embed(idx_flat, word_table, pos_table)
    return out.reshape(BATCH, SEQ, HIDDEN)


# speedup vs baseline: 1.6820x; 1.6820x over previous
import functools

import jax
import jax.numpy as jnp
from jax import lax
from jax.experimental import pallas as pl
from jax.experimental.pallas import tpu as pltpu
from jax.experimental.pallas import tpu_sc as plsc

VOCAB = 50257
HIDDEN = 768
MAX_LEN = 2048
BATCH = 4
SEQ = 2048

_INFO = plsc.get_sparse_core_info()
_NC = _INFO.num_cores
_NS = _INFO.num_subcores
_NW = _NC * _NS
_SPW = SEQ // _NW
_VECS = HIDDEN // 16


def _emb_body(idx_hbm, word_hbm, pos_hbm, out_hbm, idx_v, rows_v, pos_v, sem):
    wid = lax.axis_index("s") * _NC + lax.axis_index("c")
    s0 = wid * _SPW

    pltpu.sync_copy(pos_hbm.at[pl.ds(s0, _SPW)], pos_v)

    for b in range(BATCH):
        row0 = b * SEQ + s0
        pltpu.sync_copy(idx_hbm.at[pl.ds(row0, _SPW)], idx_v)
        pltpu.async_copy(word_hbm.at[idx_v], rows_v, sem).wait()

        def add_body(r, _, rows_v=rows_v, pos_v=pos_v):
            for j in range(_VECS):
                c = j * 16
                rows_v[r, pl.ds(c, 16)] = (
                    rows_v[r, pl.ds(c, 16)] + pos_v[r, pl.ds(c, 16)]
                )
            return _

        lax.fori_loop(0, _SPW, add_body, 0)
        pltpu.sync_copy(rows_v, out_hbm.at[pl.ds(row0, _SPW)])


@functools.partial(jax.jit, static_argnames=())
def _embed(idx_flat, word_table, pos_table):
    mesh = plsc.VectorSubcoreMesh(core_axis_name="c", subcore_axis_name="s")
    k = pl.kernel(
        _emb_body,
        out_type=jax.ShapeDtypeStruct((BATCH * SEQ, HIDDEN), jnp.float32),
        mesh=mesh,
        scratch_types=[
            pltpu.VMEM((_SPW,), jnp.int32),
            pltpu.VMEM((_SPW, HIDDEN), jnp.float32),
            pltpu.VMEM((_SPW, HIDDEN), jnp.float32),
            pltpu.SemaphoreType.DMA,
        ],
    )
    return k(idx_flat, word_table, pos_table)


def kernel(indices, word_table, pos_table):
    idx_flat = indices.reshape(-1)
    out = _embed(idx_flat, word_table, pos_table)
    return out.reshape(BATCH, SEQ, HIDDEN)
